# baseline (speedup 1.0000x reference)
import jax
import jax.numpy as jnp
from jax.experimental import pallas as pl
from jax.experimental.pallas import tpu as pltpu

_T = 512
_EPS = 1e-4


def _hnet_body(x_ref, wq_ref, wk_ref, wres_ref, bres_ref, o_ref, carry_ref):
    i = pl.program_id(1)
    T = x_ref.shape[1]
    D = x_ref.shape[2]

    xb = x_ref[0]
    qq = jnp.dot(xb, wq_ref[...], preferred_element_type=jnp.float32)
    kk = jnp.dot(xb, wk_ref[...], preferred_element_type=jnp.float32)
    rr = jnp.dot(xb, wres_ref[...], preferred_element_type=jnp.float32)
    rr = rr + bres_ref[...]

    @pl.when(i == 0)
    def _():
        carry_ref[...] = jnp.zeros_like(carry_ref)

    z_carry = carry_ref[0:1, :]
    q_carry = carry_ref[1:2, :]

    row = jax.lax.broadcasted_iota(jnp.int32, (T, 1), 0)

    q_shift = jnp.where(row == 0, q_carry, pltpu.roll(qq, 1, axis=0))

    qn2 = jnp.sum(q_shift * q_shift, axis=1, keepdims=True)
    kn2 = jnp.sum(kk * kk, axis=1, keepdims=True)
    qk = jnp.sum(q_shift * kk, axis=1, keepdims=True)
    denom = jnp.maximum(jnp.sqrt(qn2), 1e-8) * jnp.maximum(jnp.sqrt(kn2), 1e-8)
    cos = qk / denom
    p = jnp.clip(0.5 - 0.5 * cos, 0.0, 1.0)

    p = jnp.where((i == 0) & (row == 0), 1.0, p)

    sel = p >= 0.5
    p_eff = jnp.where(sel, jnp.clip(p, _EPS, 1.0 - _EPS), 0.0)

    a = 1.0 - p_eff
    bv = p_eff * xb

    G = T // 8
    for d in (1, 2, 4):
        pred = (row & 7) >= d
        a_sh = jnp.where(pred, pltpu.roll(a, d, axis=0), 1.0)
        bv_sh = jnp.where(pred, pltpu.roll(bv, d, axis=0), 0.0)
        bv = a * bv_sh + bv
        a = a * a_sh
    a3 = a.reshape(G, 8, 1)
    bv3 = bv.reshape(G, 8, D)
    ag = a3[:, 7, :]
    bg = bv3[:, 7, :]
    rowg = jax.lax.broadcasted_iota(jnp.int32, (G, 1), 0)
    d = 1
    while d < G:
        pred = rowg >= d
        ag_sh = jnp.where(pred, pltpu.roll(ag, d, axis=0), 1.0)
        bg_sh = jnp.where(pred, pltpu.roll(bg, d, axis=0), 0.0)
        bg = ag * bg_sh + bg
        ag = ag * ag_sh
        d *= 2
    zin = jnp.where(rowg == 0, z_carry,
                    pltpu.roll(bg, 1, axis=0) + pltpu.roll(ag, 1, axis=0) * z_carry)
    z3 = bv3 + a3 * zin[:, None, :]
    z = z3.reshape(T, D)

    o_ref[0] = rr + z

    carry_ref[0:1, :] = z[T - 1:T, :]
    carry_ref[1:2, :] = qq[T - 1:T, :]


def kernel(x, Wq, Wk, Wres, bres):
    B, L, D = x.shape
    T = _T
    grid = (B, L // T)
    out = pl.pallas_call(
        _hnet_body,
        grid=grid,
        in_specs=[
            pl.BlockSpec((1, T, D), lambda b, i: (b, i, 0)),
            pl.BlockSpec((D, D), lambda b, i: (0, 0)),
            pl.BlockSpec((D, D), lambda b, i: (0, 0)),
            pl.BlockSpec((D, D), lambda b, i: (0, 0)),
            pl.BlockSpec((1, D), lambda b, i: (0, 0)),
        ],
        out_specs=pl.BlockSpec((1, T, D), lambda b, i: (b, i, 0)),
        out_shape=jax.ShapeDtypeStruct((B, L, D), jnp.float32),
        scratch_shapes=[pltpu.VMEM((2, D), jnp.float32)],
        compiler_params=pltpu.CompilerParams(
            dimension_semantics=("parallel", "arbitrary"),
        ),
    )(x, Wq, Wk, Wres, bres.reshape(1, D))
    return out


# speedup vs baseline: 1.2118x; 1.2118x over previous
import jax
import jax.numpy as jnp
from jax.experimental import pallas as pl
from jax.experimental.pallas import tpu as pltpu

_T = 256
_EPS = 1e-4


def _hnet_body(x_ref, wq_ref, wk_ref, wres_ref, bres_ref, o_ref, carry_ref):
    i = pl.program_id(1)
    T = x_ref.shape[1]
    D = x_ref.shape[2]

    xb = x_ref[0]
    qq = jnp.dot(xb, wq_ref[...], preferred_element_type=jnp.float32)
    kk = jnp.dot(xb, wk_ref[...], preferred_element_type=jnp.float32)
    rr = jnp.dot(xb, wres_ref[...], preferred_element_type=jnp.float32)
    rr = rr + bres_ref[...]

    @pl.when(i == 0)
    def _():
        carry_ref[...] = jnp.zeros_like(carry_ref)

    z_carry = carry_ref[0:1, :]
    q_carry = carry_ref[1:2, :]

    row = jax.lax.broadcasted_iota(jnp.int32, (T, 1), 0)

    q_shift = jnp.where(row == 0, q_carry, pltpu.roll(qq, 1, axis=0))

    qn2 = jnp.sum(q_shift * q_shift, axis=1, keepdims=True)
    kn2 = jnp.sum(kk * kk, axis=1, keepdims=True)
    qk = jnp.sum(q_shift * kk, axis=1, keepdims=True)
    denom = jnp.maximum(jnp.sqrt(qn2), 1e-8) * jnp.maximum(jnp.sqrt(kn2), 1e-8)
    cos = qk / denom
    p = jnp.clip(0.5 - 0.5 * cos, 0.0, 1.0)

    p = jnp.where((i == 0) & (row == 0), 1.0, p)

    sel = p >= 0.5
    p_eff = jnp.where(sel, jnp.clip(p, _EPS, 1.0 - _EPS), 0.0)

    a = 1.0 - p_eff
    bv = p_eff * xb

    d = 1
    while d < T:
        pred = row >= d
        a_sh = jnp.where(pred, pltpu.roll(a, d, axis=0), 1.0)
        bv_sh = jnp.where(pred, pltpu.roll(bv, d, axis=0), 0.0)
        bv = a * bv_sh + bv
        a = a * a_sh
        d *= 2

    z = bv + a * z_carry

    o_ref[0] = rr + z

    carry_ref[0:1, :] = z[T - 1:T, :]
    carry_ref[1:2, :] = qq[T - 1:T, :]


def kernel(x, Wq, Wk, Wres, bres):
    B, L, D = x.shape
    T = _T
    grid = (B, L // T)
    out = pl.pallas_call(
        _hnet_body,
        grid=grid,
        in_specs=[
            pl.BlockSpec((1, T, D), lambda b, i: (b, i, 0)),
            pl.BlockSpec((D, D), lambda b, i: (0, 0)),
            pl.BlockSpec((D, D), lambda b, i: (0, 0)),
            pl.BlockSpec((D, D), lambda b, i: (0, 0)),
            pl.BlockSpec((1, D), lambda b, i: (0, 0)),
        ],
        out_specs=pl.BlockSpec((1, T, D), lambda b, i: (b, i, 0)),
        out_shape=jax.ShapeDtypeStruct((B, L, D), jnp.float32),
        scratch_shapes=[pltpu.VMEM((2, D), jnp.float32)],
        compiler_params=pltpu.CompilerParams(
            dimension_semantics=("parallel", "arbitrary"),
        ),
    )(x, Wq, Wk, Wres, bres.reshape(1, D))
    return out


# speedup vs baseline: 1.3076x; 1.0790x over previous
import jax
import jax.numpy as jnp
from jax.experimental import pallas as pl
from jax.experimental.pallas import tpu as pltpu

_T = 256
_EPS = 1e-4


def _hnet_body(x_ref, wq_ref, wk_ref, wres_ref, bres_ref, o_ref, carry_ref):
    i = pl.program_id(1)
    T = x_ref.shape[1]
    D = x_ref.shape[2]

    xb = x_ref[0]
    xh = xb.astype(jnp.bfloat16)
    qq = jnp.dot(xb, wq_ref[...], preferred_element_type=jnp.float32)
    kk = jnp.dot(xb, wk_ref[...], preferred_element_type=jnp.float32)
    rr = jnp.dot(xh, wres_ref[...], preferred_element_type=jnp.float32)
    rr = rr + bres_ref[...]

    @pl.when(i == 0)
    def _():
        carry_ref[...] = jnp.zeros_like(carry_ref)

    z_carry = carry_ref[0:1, :]
    q_carry = carry_ref[1:2, :]

    row = jax.lax.broadcasted_iota(jnp.int32, (T, 1), 0)

    q_shift = jnp.where(row == 0, q_carry, pltpu.roll(qq, 1, axis=0))

    qn2 = jnp.sum(q_shift * q_shift, axis=1, keepdims=True)
    kn2 = jnp.sum(kk * kk, axis=1, keepdims=True)
    qk = jnp.sum(q_shift * kk, axis=1, keepdims=True)
    denom = jnp.maximum(jnp.sqrt(qn2), 1e-8) * jnp.maximum(jnp.sqrt(kn2), 1e-8)
    cos = qk / denom
    p = jnp.clip(0.5 - 0.5 * cos, 0.0, 1.0)

    p = jnp.where((i == 0) & (row == 0), 1.0, p)

    sel = p >= 0.5
    p_eff = jnp.where(sel, jnp.clip(p, _EPS, 1.0 - _EPS), 0.0)

    a = (1.0 - p_eff).astype(jnp.bfloat16)
    bv = p_eff.astype(jnp.bfloat16) * xh
    one = jnp.bfloat16(1.0)
    zero = jnp.bfloat16(0.0)

    d = 1
    while d < T:
        pred = row >= d
        a_sh = jnp.where(pred, pltpu.roll(a, d, axis=0), one)
        bv_sh = jnp.where(pred, pltpu.roll(bv, d, axis=0), zero)
        bv = a * bv_sh + bv
        a = a * a_sh
        d *= 2

    z = bv.astype(jnp.float32) + a.astype(jnp.float32) * z_carry

    o_ref[0] = rr + z

    carry_ref[0:1, :] = z[T - 1:T, :]
    carry_ref[1:2, :] = qq[T - 1:T, :]


def kernel(x, Wq, Wk, Wres, bres):
    B, L, D = x.shape
    T = _T
    grid = (B, L // T)
    out = pl.pallas_call(
        _hnet_body,
        grid=grid,
        in_specs=[
            pl.BlockSpec((1, T, D), lambda b, i: (b, i, 0)),
            pl.BlockSpec((D, D), lambda b, i: (0, 0)),
            pl.BlockSpec((D, D), lambda b, i: (0, 0)),
            pl.BlockSpec((D, D), lambda b, i: (0, 0)),
            pl.BlockSpec((1, D), lambda b, i: (0, 0)),
        ],
        out_specs=pl.BlockSpec((1, T, D), lambda b, i: (b, i, 0)),
        out_shape=jax.ShapeDtypeStruct((B, L, D), jnp.float32),
        scratch_shapes=[pltpu.VMEM((2, D), jnp.float32)],
        compiler_params=pltpu.CompilerParams(
            dimension_semantics=("parallel", "arbitrary"),
        ),
    )(x, Wq, Wk, Wres.astype(jnp.bfloat16), bres.reshape(1, D))
    return out


# speedup vs baseline: 1.3478x; 1.0307x over previous
import jax
import jax.numpy as jnp
from jax.experimental import pallas as pl
from jax.experimental.pallas import tpu as pltpu

_T = 256
_EPS = 1e-4


def _hnet_body(x_ref, wq_ref, wk_ref, wres_ref, bres_ref, o_ref, carry_ref):
    i = pl.program_id(1)
    T = x_ref.shape[1]
    D = x_ref.shape[2]

    xb = x_ref[0]
    xh = xb.astype(jnp.bfloat16)
    qq = jnp.dot(xb, wq_ref[...], preferred_element_type=jnp.float32)
    kk = jnp.dot(xb, wk_ref[...], preferred_element_type=jnp.float32)
    rr = jnp.dot(xh, wres_ref[...], preferred_element_type=jnp.float32)
    rr = rr + bres_ref[...]

    @pl.when(i == 0)
    def _():
        carry_ref[...] = jnp.zeros_like(carry_ref)

    z_carry = carry_ref[0:1, :]
    q_carry = carry_ref[1:2, :]

    row = jax.lax.broadcasted_iota(jnp.int32, (T, 1), 0)

    q_shift = jnp.where(row == 0, q_carry, pltpu.roll(qq, 1, axis=0))

    qn2 = jnp.sum(q_shift * q_shift, axis=1, keepdims=True)
    kn2 = jnp.sum(kk * kk, axis=1, keepdims=True)
    qk = jnp.sum(q_shift * kk, axis=1, keepdims=True)
    denom = jnp.maximum(jnp.sqrt(qn2), 1e-8) * jnp.maximum(jnp.sqrt(kn2), 1e-8)
    cos = qk / denom
    p = jnp.clip(0.5 - 0.5 * cos, 0.0, 1.0)

    p = jnp.where((i == 0) & (row == 0), 1.0, p)

    sel = p >= 0.5
    p_eff = jnp.where(sel, jnp.clip(p, _EPS, 1.0 - _EPS), 0.0)

    a = (1.0 - p_eff).astype(jnp.bfloat16)
    bv = p_eff.astype(jnp.bfloat16) * xh
    one = jnp.bfloat16(1.0)
    zero = jnp.bfloat16(0.0)

    d = 1
    while d < T:
        pred = row >= d
        am = jnp.where(pred, a, zero)
        bv = am * pltpu.roll(bv, d, axis=0) + bv
        a = a * jnp.where(pred, pltpu.roll(a, d, axis=0), one)
        d *= 2

    z = bv.astype(jnp.float32) + a.astype(jnp.float32) * z_carry

    o_ref[0] = rr + z

    carry_ref[0:1, :] = z[T - 1:T, :]
    carry_ref[1:2, :] = qq[T - 1:T, :]


def kernel(x, Wq, Wk, Wres, bres):
    B, L, D = x.shape
    T = _T
    grid = (B, L // T)
    out = pl.pallas_call(
        _hnet_body,
        grid=grid,
        in_specs=[
            pl.BlockSpec((1, T, D), lambda b, i: (b, i, 0)),
            pl.BlockSpec((D, D), lambda b, i: (0, 0)),
            pl.BlockSpec((D, D), lambda b, i: (0, 0)),
            pl.BlockSpec((D, D), lambda b, i: (0, 0)),
            pl.BlockSpec((1, D), lambda b, i: (0, 0)),
        ],
        out_specs=pl.BlockSpec((1, T, D), lambda b, i: (b, i, 0)),
        out_shape=jax.ShapeDtypeStruct((B, L, D), jnp.float32),
        scratch_shapes=[pltpu.VMEM((2, D), jnp.float32)],
        compiler_params=pltpu.CompilerParams(
            dimension_semantics=("parallel", "arbitrary"),
        ),
    )(x, Wq, Wk, Wres.astype(jnp.bfloat16), bres.reshape(1, D))
    return out


# speedup vs baseline: 1.3839x; 1.0268x over previous
import jax
import jax.numpy as jnp
from jax.experimental import pallas as pl
from jax.experimental.pallas import tpu as pltpu

_T = 512
_EPS = 1e-4


def _hnet_body(x_ref, wq_ref, wk_ref, wres_ref, bres_ref, o_ref, carry_ref):
    i = pl.program_id(1)
    T = x_ref.shape[1]
    D = x_ref.shape[2]

    xb = x_ref[0]
    xh = xb.astype(jnp.bfloat16)
    qq = jnp.dot(xb, wq_ref[...], preferred_element_type=jnp.float32)
    kk = jnp.dot(xb, wk_ref[...], preferred_element_type=jnp.float32)
    rr = jnp.dot(xh, wres_ref[...], preferred_element_type=jnp.float32)
    rr = rr + bres_ref[...]

    @pl.when(i == 0)
    def _():
        carry_ref[...] = jnp.zeros_like(carry_ref)

    z_carry = carry_ref[0:1, :]
    q_carry = carry_ref[1:2, :]

    row = jax.lax.broadcasted_iota(jnp.int32, (T, 1), 0)

    q_shift = jnp.where(row == 0, q_carry, pltpu.roll(qq, 1, axis=0))

    qn2 = jnp.sum(q_shift * q_shift, axis=1, keepdims=True)
    kn2 = jnp.sum(kk * kk, axis=1, keepdims=True)
    qk = jnp.sum(q_shift * kk, axis=1, keepdims=True)
    denom = jnp.maximum(jnp.sqrt(qn2), 1e-8) * jnp.maximum(jnp.sqrt(kn2), 1e-8)
    cos = qk / denom
    p = jnp.clip(0.5 - 0.5 * cos, 0.0, 1.0)

    p = jnp.where((i == 0) & (row == 0), 1.0, p)

    sel = p >= 0.5
    p_eff = jnp.where(sel, jnp.clip(p, _EPS, 1.0 - _EPS), 0.0)

    a = (1.0 - p_eff).astype(jnp.bfloat16)
    bv = p_eff.astype(jnp.bfloat16) * xh
    one = jnp.bfloat16(1.0)
    zero = jnp.bfloat16(0.0)

    d = 1
    while d < T:
        pred = row >= d
        am = jnp.where(pred, a, zero)
        bv = am * pltpu.roll(bv, d, axis=0) + bv
        a = a * jnp.where(pred, pltpu.roll(a, d, axis=0), one)
        d *= 2

    z = bv.astype(jnp.float32) + a.astype(jnp.float32) * z_carry

    o_ref[0] = rr + z

    carry_ref[0:1, :] = z[T - 1:T, :]
    carry_ref[1:2, :] = qq[T - 1:T, :]


def kernel(x, Wq, Wk, Wres, bres):
    B, L, D = x.shape
    T = _T
    grid = (B, L // T)
    out = pl.pallas_call(
        _hnet_body,
        grid=grid,
        in_specs=[
            pl.BlockSpec((1, T, D), lambda b, i: (b, i, 0)),
            pl.BlockSpec((D, D), lambda b, i: (0, 0)),
            pl.BlockSpec((D, D), lambda b, i: (0, 0)),
            pl.BlockSpec((D, D), lambda b, i: (0, 0)),
            pl.BlockSpec((1, D), lambda b, i: (0, 0)),
        ],
        out_specs=pl.BlockSpec((1, T, D), lambda b, i: (b, i, 0)),
        out_shape=jax.ShapeDtypeStruct((B, L, D), jnp.float32),
        scratch_shapes=[pltpu.VMEM((2, D), jnp.float32)],
        compiler_params=pltpu.CompilerParams(
            dimension_semantics=("parallel", "arbitrary"),
        ),
    )(x, Wq, Wk, Wres.astype(jnp.bfloat16), bres.reshape(1, D))
    return out


# speedup vs baseline: 1.3848x; 1.0006x over previous
import jax
import jax.numpy as jnp
from jax.experimental import pallas as pl
from jax.experimental.pallas import tpu as pltpu

_T = 1024
_EPS = 1e-4


def _hnet_body(x_ref, wq_ref, wk_ref, wres_ref, bres_ref, o_ref, carry_ref):
    i = pl.program_id(1)
    T = x_ref.shape[1]
    D = x_ref.shape[2]

    xb = x_ref[0]
    xh = xb.astype(jnp.bfloat16)
    qq = jnp.dot(xb, wq_ref[...], preferred_element_type=jnp.float32)
    kk = jnp.dot(xb, wk_ref[...], preferred_element_type=jnp.float32)
    rr = jnp.dot(xh, wres_ref[...], preferred_element_type=jnp.float32)
    rr = rr + bres_ref[...]

    @pl.when(i == 0)
    def _():
        carry_ref[...] = jnp.zeros_like(carry_ref)

    z_carry = carry_ref[0:1, :]
    q_carry = carry_ref[1:2, :]

    row = jax.lax.broadcasted_iota(jnp.int32, (T, 1), 0)

    q_shift = jnp.where(row == 0, q_carry, pltpu.roll(qq, 1, axis=0))

    qn2 = jnp.sum(q_shift * q_shift, axis=1, keepdims=True)
    kn2 = jnp.sum(kk * kk, axis=1, keepdims=True)
    qk = jnp.sum(q_shift * kk, axis=1, keepdims=True)
    denom = jnp.maximum(jnp.sqrt(qn2), 1e-8) * jnp.maximum(jnp.sqrt(kn2), 1e-8)
    cos = qk / denom
    p = jnp.clip(0.5 - 0.5 * cos, 0.0, 1.0)

    p = jnp.where((i == 0) & (row == 0), 1.0, p)

    sel = p >= 0.5
    p_eff = jnp.where(sel, jnp.clip(p, _EPS, 1.0 - _EPS), 0.0)

    a = (1.0 - p_eff).astype(jnp.bfloat16)
    bv = p_eff.astype(jnp.bfloat16) * xh
    one = jnp.bfloat16(1.0)
    zero = jnp.bfloat16(0.0)

    d = 1
    while d < T:
        pred = row >= d
        am = jnp.where(pred, a, zero)
        bv = am * pltpu.roll(bv, d, axis=0) + bv
        a = a * jnp.where(pred, pltpu.roll(a, d, axis=0), one)
        d *= 2

    z = bv.astype(jnp.float32) + a.astype(jnp.float32) * z_carry

    o_ref[0] = rr + z

    carry_ref[0:1, :] = z[T - 1:T, :]
    carry_ref[1:2, :] = qq[T - 1:T, :]


def kernel(x, Wq, Wk, Wres, bres):
    B, L, D = x.shape
    T = _T
    grid = (B, L // T)
    out = pl.pallas_call(
        _hnet_body,
        grid=grid,
        in_specs=[
            pl.BlockSpec((1, T, D), lambda b, i: (b, i, 0)),
            pl.BlockSpec((D, D), lambda b, i: (0, 0)),
            pl.BlockSpec((D, D), lambda b, i: (0, 0)),
            pl.BlockSpec((D, D), lambda b, i: (0, 0)),
            pl.BlockSpec((1, D), lambda b, i: (0, 0)),
        ],
        out_specs=pl.BlockSpec((1, T, D), lambda b, i: (b, i, 0)),
        out_shape=jax.ShapeDtypeStruct((B, L, D), jnp.float32),
        scratch_shapes=[pltpu.VMEM((2, D), jnp.float32)],
        compiler_params=pltpu.CompilerParams(
            dimension_semantics=("parallel", "arbitrary"),
        ),
    )(x, Wq, Wk, Wres.astype(jnp.bfloat16), bres.reshape(1, D))
    return out
